# baseline (speedup 1.0000x reference)
import functools

import jax
import jax.numpy as jnp
from jax import lax
from jax.experimental import pallas as pl
from jax.experimental.pallas import tpu as pltpu
from jax.experimental.pallas import tpu_sc as plsc

_NC = 2
_NS = 16
_NW = _NC * _NS
_CHUNK = 128


@functools.lru_cache(maxsize=None)
def _make_gather(batch, nflat):
  bpw = batch // _NW
  nchunk = bpw // _CHUNK
  mesh = plsc.VectorSubcoreMesh(core_axis_name="c", subcore_axis_name="s")

  @functools.partial(
      pl.kernel,
      mesh=mesh,
      out_type=jax.ShapeDtypeStruct((_NW, nchunk, _CHUNK), jnp.float32),
      compiler_params=pltpu.CompilerParams(
          disable_bounds_checks=True, disable_semaphore_checks=True),
      scratch_types=[
          pltpu.VMEM((nchunk, _CHUNK), jnp.int32),
          pltpu.VMEM((nchunk, _CHUNK), jnp.float32),
          pltpu.SemaphoreType.DMA,
          pltpu.SemaphoreType.DMA,
          pltpu.SemaphoreType.DMA,
      ],
  )
  def gather_kernel(idx_hbm, table_hbm, out_hbm, idx_v, rows_v, isem, gsem,
                    wsem):
    wid = lax.axis_index("s") * _NC + lax.axis_index("c")
    pltpu.async_copy(idx_hbm.at[wid], idx_v, isem).wait()
    gathers = [
        pltpu.async_copy(table_hbm.at[idx_v.at[j]], rows_v.at[j], gsem)
        for j in range(nchunk)
    ]
    for g in gathers:
      g.wait()
    pltpu.async_copy(rows_v, out_hbm.at[wid], wsem).wait()

  return gather_kernel


def kernel(e, table):
  batch = e.shape[0]
  nrows = table.shape[0]
  bpw = batch // _NW
  nchunk = bpw // _CHUNK
  idx = e.astype(jnp.int32).reshape(_NW, nchunk, _CHUNK)
  pad = (-nrows) % 1024
  flat = jnp.pad(table, ((0, pad), (0, 0))).reshape(nrows + pad)
  out = _make_gather(batch, nrows + pad)(idx, flat)
  return out.reshape(batch, 1)


# speedup vs baseline: 1.0046x; 1.0046x over previous
import functools

import jax
import jax.numpy as jnp
from jax import lax
from jax.experimental import pallas as pl
from jax.experimental.pallas import tpu as pltpu
from jax.experimental.pallas import tpu_sc as plsc

_NC = 2
_NS = 16
_NW = _NC * _NS
_CHUNK = 128


@functools.lru_cache(maxsize=None)
def _make_gather(batch, nflat):
  bpw = batch // _NW
  nchunk = bpw // _CHUNK
  mesh = plsc.VectorSubcoreMesh(core_axis_name="c", subcore_axis_name="s")

  @functools.partial(
      pl.kernel,
      mesh=mesh,
      out_type=jax.ShapeDtypeStruct((_NW, nchunk, _CHUNK), jnp.float32),
      scratch_types=[
          pltpu.VMEM((nchunk, _CHUNK), jnp.int32),
          pltpu.VMEM((nchunk, _CHUNK), jnp.float32),
          pltpu.SemaphoreType.DMA,
          pltpu.SemaphoreType.DMA,
          pltpu.SemaphoreType.DMA,
      ],
  )
  def gather_kernel(idx_hbm, table_hbm, out_hbm, idx_v, rows_v, isem, gsem,
                    wsem):
    wid = lax.axis_index("s") * _NC + lax.axis_index("c")
    pltpu.async_copy(idx_hbm.at[wid], idx_v, isem).wait()
    gathers = [
        pltpu.async_copy(table_hbm.at[idx_v.at[j]], rows_v.at[j], gsem)
        for j in range(nchunk)
    ]
    for g in gathers:
      g.wait()
    pltpu.async_copy(rows_v, out_hbm.at[wid], wsem).wait()

  return gather_kernel


def kernel(e, table):
  batch = e.shape[0]
  nrows = table.shape[0]
  bpw = batch // _NW
  nchunk = bpw // _CHUNK
  idx = e.astype(jnp.int32).reshape(_NW, nchunk, _CHUNK)
  pad = (-nrows) % 1024
  flat = jnp.pad(table, ((0, pad), (0, 0))).reshape(nrows + pad)
  out = _make_gather(batch, nrows + pad)(idx, flat)
  return out.reshape(batch, 1)
